# baseline (speedup 1.0000x reference)
import jax
import jax.numpy as jnp
from jax.experimental import pallas as pl
from jax.experimental.pallas import tpu as pltpu

_HBM = pltpu.MemorySpace.HBM


def _gnn_body(nb, n, d_in, d_edge, d, n_layers,
              a_ref, e_ref, p_ref, out_ref, ev, sem):
    copies = []
    for b in range(nb):
        c = pltpu.make_async_copy(e_ref.at[b], ev.at[b], sem)
        c.start()
        copies.append(c)

    r = 0
    wn = p_ref[r:r + d_in, 0:d]; r += d_in
    bn = p_ref[r:r + 1, 0:d]; r += 1
    we = p_ref[r:r + d_edge, 0:d]; r += d_edge
    be = p_ref[r:r + 1, 0:d]; r += 1
    wc0 = r; r += n_layers * d
    bc0 = r; r += n_layers
    epsr = r; r += 1
    wo = p_ref[r:r + 1, 0:d]; r += 1
    bo = p_ref[r:r + 1, 0:1]; r += 1
    x0 = r

    for b in range(nb):
        a = a_ref[b]
        x = p_ref[x0 + b * n: x0 + (b + 1) * n, 0:d_in]
        h = jnp.dot(x, wn, preferred_element_type=jnp.float32) + bn
        copies[b].wait()
        eb = ev[b]
        if d_edge == 1:
            ms = [a * eb]
        else:
            ms = [a * eb[:, k::d_edge] for k in range(d_edge)]
        for l in range(n_layers):
            ah = jnp.dot(a, h, preferred_element_type=jnp.float32)
            msg = be * ah
            for k in range(d_edge):
                mh = jnp.dot(ms[k], h, preferred_element_type=jnp.float32)
                msg = msg + we[k:k + 1, :] * mh
            wc = p_ref[wc0 + l * d: wc0 + (l + 1) * d, 0:d]
            bc = p_ref[bc0 + l: bc0 + l + 1, 0:d]
            pre = jnp.dot((1.0 + p_ref[epsr, l]) * h + msg, wc,
                          preferred_element_type=jnp.float32) + bc
            h = jnp.maximum(pre, 0.0)
        hm = jnp.mean(h, axis=0, keepdims=True)
        val = jnp.sum(hm * wo, axis=1, keepdims=True) + bo
        out_ref[b:b + 1, :] = 1.0 + jnp.where(val >= 0.0, val, 0.01 * val)


def kernel(A, X, E, We, be, Wn, bn, Wc, bc, eps, Wo, bo):
    nb, n, d_in = X.shape
    d_edge, d = We.shape
    n_layers = Wc.shape[0]

    def row(m, width=128):
        m = m.reshape(-1, m.shape[-1]).astype(jnp.float32)
        return jnp.pad(m, ((0, 0), (0, width - m.shape[-1])))

    pack = jnp.concatenate([
        row(Wn), row(bn.reshape(1, d)), row(We), row(be.reshape(1, d)),
        row(Wc.reshape(n_layers * d, d)), row(bc),
        row(jnp.pad(eps.reshape(1, n_layers), ((0, 0), (0, 1)))),
        row(Wo.reshape(1, d)),
        row(bo.reshape(1, 1)),
        row(X.reshape(nb * n, d_in)),
    ], axis=0)

    def body(a_ref, e_ref, p_ref, out_ref, ev, sem):
        _gnn_body(nb, n, d_in, d_edge, d, n_layers,
                  a_ref, e_ref, p_ref, out_ref, ev, sem)

    out = pl.pallas_call(
        body,
        in_specs=[
            pl.BlockSpec(),
            pl.BlockSpec(memory_space=_HBM),
            pl.BlockSpec(),
        ],
        scratch_shapes=[
            pltpu.VMEM((nb, n, n * d_edge), jnp.float32),
            pltpu.SemaphoreType.DMA,
        ],
        out_shape=jax.ShapeDtypeStruct((nb, 1), jnp.float32),
    )(A, E.reshape(nb, n, n * d_edge), pack)
    return out


# speedup vs baseline: 1.1967x; 1.1967x over previous
import jax
import jax.numpy as jnp
from jax.experimental import pallas as pl


def _gnn_body(nb, n, d_in, d_edge, d, n_layers, a_ref, e_ref, p_ref, out_ref):
    r = 0
    wn = p_ref[r:r + d_in, :]; r += d_in
    bn = p_ref[r:r + 1, :]; r += 1
    we = p_ref[r:r + d_edge, :]; r += d_edge
    be = p_ref[r:r + 1, :]; r += 1
    wc0 = r; r += n_layers * d
    bc0 = r; r += n_layers
    epsr = r; r += 1
    wo = p_ref[r:r + 1, :]; r += 1
    bo = p_ref[r:r + 1, 0:1]; r += 1
    x0 = r

    for b in range(nb):
        a = a_ref[b]
        x = p_ref[x0 + b * n: x0 + (b + 1) * n, 0:d_in]
        h = jnp.dot(x, wn, preferred_element_type=jnp.float32) + bn
        eb = e_ref[b]
        if d_edge == 1:
            ms = [a * eb]
        else:
            ms = [a * eb[:, k::d_edge] for k in range(d_edge)]
        for l in range(n_layers):
            ah = jnp.dot(a, h, preferred_element_type=jnp.float32)
            msg = be * ah
            for k in range(d_edge):
                mh = jnp.dot(ms[k], h, preferred_element_type=jnp.float32)
                msg = msg + we[k:k + 1, :] * mh
            wc = p_ref[wc0 + l * d: wc0 + (l + 1) * d, :]
            bc = p_ref[bc0 + l: bc0 + l + 1, :]
            pre = jnp.dot((1.0 + p_ref[epsr, l]) * h + msg, wc,
                          preferred_element_type=jnp.float32) + bc
            h = jnp.maximum(pre, 0.0)
        hm = jnp.mean(h, axis=0, keepdims=True)
        val = jnp.sum(hm * wo, axis=1, keepdims=True) + bo
        out_ref[b:b + 1, :] = 1.0 + jnp.where(val >= 0.0, val, 0.01 * val)


def kernel(A, X, E, We, be, Wn, bn, Wc, bc, eps, Wo, bo):
    nb, n, d_in = X.shape
    d_edge, d = We.shape
    n_layers = Wc.shape[0]

    def row(m):
        m = m.astype(jnp.float32)
        return jnp.pad(m, ((0, 0), (0, d - m.shape[-1])))

    pack = jnp.concatenate([
        Wn, bn.reshape(1, d), We, be.reshape(1, d),
        Wc.reshape(n_layers * d, d), bc,
        row(eps.reshape(1, n_layers)),
        Wo.reshape(1, d),
        row(bo.reshape(1, 1)),
        row(X.reshape(nb * n, d_in)),
    ], axis=0)

    def body(a_ref, e_ref, p_ref, out_ref):
        _gnn_body(nb, n, d_in, d_edge, d, n_layers,
                  a_ref, e_ref, p_ref, out_ref)

    out = pl.pallas_call(
        body,
        out_shape=jax.ShapeDtypeStruct((nb, 1), jnp.float32),
    )(A, E.reshape(nb, n, n * d_edge), pack)
    return out
